# Initial kernel scaffold; baseline (speedup 1.0000x reference)
import jax
import jax.numpy as jnp
from jax.experimental import pallas as pl


def kernel(question_embedding, object_features_list, bounding_boxes, batch_size, num_obj, edge_index, batch, W1, a1_src, a1_dst, b1, W2, a2_src, a2_dst, b2, W3, a3_src, a3_dst, b3):
    raise NotImplementedError("write your pallas kernel here")


# speedup vs baseline: 16.7307x; 16.7307x over previous
import functools

import jax
import jax.numpy as jnp
from jax import lax
from jax.experimental import pallas as pl
from jax.experimental.pallas import tpu as pltpu
from jax.experimental.pallas import tpu_sc as plsc

N = 10000
E_RAW = 160000
ETOT = N + E_RAW
NG = 10752
EPAD = NG * 16
NPAD = 10000
MB = 200
D_CHUNK = 640
NSC = 2
NTILE = 16
CHUNK_SIZES = (640, 640, 640, 640, 640, 640, 640, 520)
GB = 64
BLK_A = 112


def _leaky(x):
    return jnp.where(x > 0, x, 0.2 * x)


def _tc_matmul(x_p, W, a_s, a_d, *, K, F, CB, H):
    grid = (NPAD // MB, F // CB)
    hd_per_cb = CB // (F // H)

    def body(x_ref, w_ref, as_ref, ad_ref, xw_ref, als_ref, ald_ref):
        f = pl.program_id(1)
        xwb = jnp.dot(x_ref[...], w_ref[...],
                      preferred_element_type=jnp.float32)
        xw_ref[...] = xwb
        ps = jnp.sum(xwb * as_ref[...], axis=1, keepdims=True)
        pd = jnp.sum(xwb * ad_ref[...], axis=1, keepdims=True)
        col = lax.broadcasted_iota(jnp.int32, (MB, 128), 1)
        us = jnp.where(col == f, ps, 0.0)
        ud = jnp.where(col == f, pd, 0.0)

        @pl.when(f == 0)
        def _():
            als_ref[...] = us
            ald_ref[...] = ud

        @pl.when(f > 0)
        def _():
            als_ref[...] = als_ref[...] + us
            ald_ref[...] = ald_ref[...] + ud

    return pl.pallas_call(
        body,
        grid=grid,
        in_specs=[
            pl.BlockSpec((MB, K), lambda m, f: (m, 0)),
            pl.BlockSpec((K, CB), lambda m, f: (0, f)),
            pl.BlockSpec((1, CB), lambda m, f: (0, f)),
            pl.BlockSpec((1, CB), lambda m, f: (0, f)),
        ],
        out_specs=[
            pl.BlockSpec((MB, CB), lambda m, f: (m, f)),
            pl.BlockSpec((MB, 128), lambda m, f: (m, 0)),
            pl.BlockSpec((MB, 128), lambda m, f: (m, 0)),
        ],
        out_shape=[
            jax.ShapeDtypeStruct((NPAD, F), jnp.float32),
            jax.ShapeDtypeStruct((NPAD, 128), jnp.float32),
            jax.ShapeDtypeStruct((NPAD, 128), jnp.float32),
        ],
        compiler_params=pltpu.CompilerParams(
            dimension_semantics=("parallel", "arbitrary")),
    )(x_p, W.reshape(K, F), a_s.reshape(1, F), a_d.reshape(1, F))


def _sc_edge_logits(sd2d, als_flat, ald_flat, *, H):
    EXW = 16 * H
    NH = N * H
    g_per_tile = NG // (NSC * NTILE)
    nblk = g_per_tile // BLK_A
    mesh = plsc.VectorSubcoreMesh(core_axis_name="c", subcore_axis_name="s")

    @functools.partial(
        pl.kernel,
        out_type=jax.ShapeDtypeStruct((NG, EXW), jnp.float32),
        mesh=mesh,
        compiler_params=pltpu.CompilerParams(needs_layout_passes=False),
        scratch_types=[
            pltpu.VMEM((NH,), jnp.float32),
            pltpu.VMEM((NH,), jnp.float32),
            pltpu.VMEM((BLK_A, 32), jnp.int32),
            pltpu.VMEM((BLK_A, EXW), jnp.float32),
        ],
    )
    def k(sd_h, als_h, ald_h, ex_h, als_v, ald_v, sd_v, ex_v):
        c = lax.axis_index("c")
        s = lax.axis_index("s")
        wid = c * NTILE + s
        pltpu.sync_copy(als_h, als_v)
        pltpu.sync_copy(ald_h, ald_v)
        iota = lax.iota(jnp.int32, 16)

        for b in range(nblk):
            g0 = wid * g_per_tile + b * BLK_A
            pltpu.sync_copy(sd_h.at[pl.ds(g0, BLK_A)], sd_v)

            def g_body(gl, _):
                src16 = sd_v[gl, pl.ds(0, 16)]
                dst16 = sd_v[gl, pl.ds(16, 16)]
                eid = (g0 + gl) * 16 + iota
                valid = eid < ETOT
                for h in range(H):
                    gs = plsc.load_gather(als_v, [src16 * H + h])
                    gd = plsc.load_gather(ald_v, [dst16 * H + h])
                    ev = jnp.exp(_leaky(gs + gd))
                    ev = jnp.where(valid, ev, 0.0)
                    ex_v[gl, pl.ds(h * 16, 16)] = ev
                return 0

            lax.fori_loop(0, BLK_A, g_body, 0)
            pltpu.sync_copy(ex_v, ex_h.at[pl.ds(g0, BLK_A)])

    return k(sd2d, als_flat, ald_flat)


TS = 320


def _sc_aggregate(xw, edata, cptr, bias, skip, *,
                  F, H, D_T, NC, mode):
    EXW = 16 * H
    C = F // H
    CS = C // 16
    FS = F // 16
    FOUT = C if mode == "mean" else F
    RB = 8
    GBL = GB if mode != "mean" else 8
    CLOC = ((NC + 1 + 7) // 8) * 8 + 8
    EDW = 32 + EXW
    mesh = plsc.VectorSubcoreMesh(core_axis_name="c", subcore_axis_name="s")

    scratch = [
        pltpu.VMEM((D_T, F), jnp.float32),
        pltpu.VMEM((D_T * 16,), jnp.float32),
        pltpu.VMEM((CLOC,), jnp.int32),
        pltpu.VMEM((GBL + 8, EDW), jnp.int32),
        pltpu.VMEM((16, F), jnp.float32),
        pltpu.VMEM((16, F), jnp.float32),
        pltpu.VMEM((16,), jnp.int32),
        pltpu.VMEM((16,), jnp.int32),
        pltpu.VMEM((RB, FOUT), jnp.float32),
        pltpu.VMEM((FOUT,), jnp.float32),
        pltpu.SemaphoreType.DMA,
        pltpu.SemaphoreType.DMA,
    ]

    @functools.partial(
        pl.kernel,
        out_type=jax.ShapeDtypeStruct((N, FOUT), jnp.float32),
        mesh=mesh,
        compiler_params=pltpu.CompilerParams(needs_layout_passes=False),
        scratch_types=scratch,
    )
    def k(xw_h, ed_h, cptr_h, bias_h, skip_h, out_h,
          acc_v, den_v, cptr_v, ed_v, rows0_v, rows1_v,
          idx0_v, idx1_v, obuf_v, bias_v, sem0, sem1):
        c = lax.axis_index("c")
        s = lax.axis_index("s")
        w = c * NTILE + s
        ca = ((w * NC) // 8) * 8
        pltpu.sync_copy(cptr_h.at[pl.ds(ca, CLOC)], cptr_v)
        pltpu.sync_copy(bias_h, bias_v)
        iota = lax.iota(jnp.int32, 16)
        bufs = ((idx0_v, rows0_v, sem0), (idx1_v, rows1_v, sem1))

        def sget(i):
            return jnp.max(plsc.load_gather(
                cptr_v, [jnp.full((16,), i, jnp.int32)]))

        def chunk_body(ci, _):
            base_c = w * TS + ci * D_T

            def zbody(r, _):
                for j in range(FS):
                    acc_v[r, pl.ds(j * 16, 16)] = jnp.zeros((16,), jnp.float32)
                den_v[pl.ds(r * 16, 16)] = jnp.zeros((16,), jnp.float32)
                return 0

            lax.fori_loop(0, D_T, zbody, 0)

            e_lo = sget(w * NC - ca + ci)
            e_hi = sget(w * NC - ca + ci + 1)
            G_lo = e_lo // 16
            G_hi = (e_hi + 15) // 16
            nblk = (G_hi - G_lo + GBL - 1) // GBL

            def blk_body(b, _):
                gb0 = G_lo + b * GBL
                gb1 = jnp.minimum(gb0 + GBL, G_hi)
                gb0a = (gb0 // 8) * 8
                pltpu.sync_copy(ed_h.at[pl.ds(gb0a, GBL + 8)], ed_v)

                def issue(g, idxr, rowsr, semr):
                    @pl.when(g < gb1)
                    def _():
                        idxr[:] = ed_v[g - gb0a, pl.ds(0, 16)]
                        pltpu.async_copy(xw_h.at[idxr], rowsr, semr)

                issue(gb0, *bufs[0])
                issue(gb0 + 1, *bufs[1])

                def process(g, idxr, rowsr, semr):
                    gl = g - gb0a
                    pltpu.make_async_copy(xw_h.at[idxr], rowsr, semr).wait()

                    def e_body(i, _):
                        f16gl = jnp.full((16,), gl, jnp.int32)
                        dstb = plsc.load_gather(
                            ed_v, [f16gl, jnp.full((16,), 16 + i, jnp.int32)])
                        row = jnp.clip(jnp.max(dstb) - base_c, 0, D_T - 1)
                        eid = g * 16 + i
                        vok = jnp.logical_and(
                            eid >= e_lo, eid < e_hi).astype(jnp.float32)
                        denvec = jnp.zeros((16,), jnp.float32)
                        for h in range(H):
                            exb = plsc.bitcast(plsc.load_gather(
                                ed_v,
                                [f16gl,
                                 jnp.full((16,), 32 + h * 16 + i, jnp.int32)]),
                                jnp.float32)
                            exb = exb * vok
                            denvec = jnp.where(iota == h, exb, denvec)
                            offs = [h * C + j * 16 for j in range(CS)]
                            vals = [rowsr[i, pl.ds(o, 16)] for o in offs]
                            vals = [v * exb for v in vals]
                            for o, v in zip(offs, vals):
                                plsc.addupdate(
                                    acc_v.at[row, pl.ds(o, 16)], v)
                        plsc.addupdate(den_v.at[pl.ds(row * 16, 16)], denvec)
                        return 0

                    lax.fori_loop(0, 16, e_body, 0)
                    issue(g + 2, idxr, rowsr, semr)

                def pair_body(q, _):
                    g = gb0 + 2 * q
                    for p in range(2):
                        gp = g + p

                        @pl.when(gp < gb1)
                        def _(gp=gp, p=p):
                            process(gp, *bufs[p])

                    return 0

                lax.fori_loop(0, (gb1 - gb0 + 1) // 2, pair_body, 0)
                return 0

            lax.fori_loop(0, nblk, blk_body, 0)

            def nrm_body(kb, _):
                r8 = kb * RB

                @pl.when(base_c + r8 + RB <= N)
                def _():
                    if mode == "skip":
                        pltpu.sync_copy(
                            skip_h.at[pl.ds(base_c + r8, RB)], obuf_v)

                    def row_body(r, _):
                        ar = r8 + r
                        if mode == "mean":
                            invs = []
                            for h in range(H):
                                denb = plsc.load_gather(
                                    den_v,
                                    [jnp.full((16,), ar * 16 + h, jnp.int32)])
                                invs.append(1.0 / ((denb + 1e-16) * H))
                            for j in range(CS):
                                a16 = bias_v[pl.ds(j * 16, 16)]
                                for h in range(H):
                                    a16 = a16 + (
                                        acc_v[ar, pl.ds(h * C + j * 16, 16)]
                                        * invs[h])
                                obuf_v[r, pl.ds(j * 16, 16)] = a16
                        else:
                            for h in range(H):
                                denb = plsc.load_gather(
                                    den_v,
                                    [jnp.full((16,), ar * 16 + h, jnp.int32)])
                                inv = 1.0 / (denb + 1e-16)
                                for j in range(CS):
                                    off = h * C + j * 16
                                    v = (acc_v[ar, pl.ds(off, 16)] * inv
                                         + bias_v[pl.ds(off, 16)])
                                    v = jnp.maximum(v, 0.0)
                                    if mode == "skip":
                                        v = v + obuf_v[r, pl.ds(off, 16)]
                                    obuf_v[r, pl.ds(off, 16)] = v
                        return 0

                    lax.fori_loop(0, RB, row_body, 0)
                    pltpu.sync_copy(obuf_v, out_h.at[pl.ds(base_c + r8, RB)])

                return 0

            lax.fori_loop(0, D_T // RB, nrm_body, 0)
            return 0

        lax.fori_loop(0, NC, chunk_body, 0)

    if skip is None:
        skip = jnp.zeros((N, 8), jnp.float32)
    return k(xw, edata, cptr, bias, skip)


def _make_cptr(dst_sorted, D_T, NC):
    bounds = [min(w * TS + ci * D_T, N)
              for w in range(NSC * NTILE) for ci in range(NC)] + [N]
    cptr = jnp.searchsorted(
        dst_sorted, jnp.asarray(bounds, jnp.int32), side="left"
    ).astype(jnp.int32)
    cloc = ((NC + 1 + 7) // 8) * 8 + 8
    cpad = ((31 * NC + cloc + 7) // 8) * 8 + 8
    return jnp.full((cpad,), ETOT, jnp.int32).at[:len(bounds)].set(cptr)


def _layer(x_p, W, a_s, a_d, bias, sd2d, cptr, skip, *,
           K, F, H, D_T, NC, mode):
    CB = F // H
    xw, als_p, ald_p = _tc_matmul(x_p, W, a_s, a_d, K=K, F=F, CB=CB, H=H)
    als_flat = als_p[:N, :H].reshape(-1)
    ald_flat = ald_p[:N, :H].reshape(-1)
    ex2d = _sc_edge_logits(sd2d, als_flat, ald_flat, H=H)
    edata = jnp.concatenate(
        [sd2d, lax.bitcast_convert_type(ex2d, jnp.int32)], axis=1)
    return _sc_aggregate(xw, edata, cptr, bias, skip,
                         F=F, H=H, D_T=D_T, NC=NC, mode=mode)


def kernel(question_embedding, object_features_list, bounding_boxes,
           batch_size, num_obj, edge_index, batch, W1, a1_src, a1_dst, b1,
           W2, a2_src, a2_dst, b2, W3, a3_src, a3_dst, b3):
    loops = jnp.arange(N, dtype=edge_index.dtype)
    src = jnp.concatenate([edge_index[0], loops])
    dst = jnp.concatenate([edge_index[1], loops])
    order = jnp.argsort(dst)
    dst_sorted = dst[order]
    src_s = jnp.zeros((EPAD,), jnp.int32).at[:ETOT].set(src[order])
    dst_s = jnp.zeros((EPAD,), jnp.int32).at[:ETOT].set(dst_sorted)
    sd2d = jnp.concatenate(
        [src_s.reshape(NG, 16), dst_s.reshape(NG, 16)], axis=1)

    cptr12 = _make_cptr(dst_sorted, 64, 5)
    cptr3 = _make_cptr(dst_sorted, 8, 40)

    x1 = jnp.concatenate([question_embedding, object_features_list], axis=-1)

    g1 = _layer(x1, W1, a1_src, a1_dst, b1, sd2d, cptr12, None,
                K=2048, F=1024, H=4, D_T=64, NC=5, mode="relu")
    g2 = _layer(g1, W2, a2_src, a2_dst, b2, sd2d, cptr12, g1,
                K=1024, F=1024, H=4, D_T=64, NC=5, mode="skip")
    out = _layer(g2, W3, a3_src, a3_dst, b3, sd2d, cptr3, None,
                 K=1024, F=2560, H=5, D_T=8, NC=40, mode="mean")
    return out
